# Initial kernel scaffold; baseline (speedup 1.0000x reference)
import functools

import jax
import jax.numpy as jnp
from jax import lax
from jax.experimental import pallas as pl
from jax.experimental.pallas import tpu as pltpu
from jax.experimental.pallas import tpu_sc as plsc

_N = 10000
_E = 320000
_D = 128
_NC = 2
_NS = 16
_NW = _NC * _NS
_EPW = _E // _NW
_C = 80
_K = _EPW // _C
_RPT = _N // _NS
_CW = 16

_PREC = lax.Precision.HIGHEST


def _sc_agg_body(table, src, dst, zrow, zcnt, ones,
                 out_acc, out_cnt, src_v, dst_v, rows_v, ones_v,
                 acc_sh, cnt_sh, sem):
    c = lax.axis_index("c")
    s = lax.axis_index("s")
    pltpu.sync_copy(zrow, acc_sh.at[pl.ds(s * _RPT, _RPT)])
    pltpu.sync_copy(zcnt, cnt_sh.at[pl.ds(s * _RPT, _RPT)])
    pltpu.sync_copy(ones, ones_v)
    plsc.subcore_barrier()

    base = (c * _NS + s) * _EPW

    def chunk(k, carry):
        off = base + k * _C
        pltpu.sync_copy(src.at[pl.ds(off, _C)], src_v)
        pltpu.sync_copy(dst.at[pl.ds(off, _C)], dst_v)
        pltpu.async_copy(table.at[src_v], rows_v, sem).wait()
        pltpu.sync_copy(rows_v, acc_sh.at[dst_v], add=True)
        pltpu.sync_copy(ones_v, cnt_sh.at[dst_v], add=True)
        return carry

    lax.fori_loop(0, _K, chunk, 0)
    plsc.subcore_barrier()
    pltpu.sync_copy(acc_sh.at[pl.ds(s * _RPT, _RPT)],
                    out_acc.at[c, pl.ds(s * _RPT, _RPT)])
    pltpu.sync_copy(cnt_sh.at[pl.ds(s * _RPT, _RPT)],
                    out_cnt.at[c, pl.ds(s * _RPT, _RPT)])


_sc_agg = functools.partial(
    pl.kernel,
    out_type=(jax.ShapeDtypeStruct((_NC, _N, _D), jnp.float32),
              jax.ShapeDtypeStruct((_NC, _N, _CW), jnp.float32)),
    mesh=plsc.VectorSubcoreMesh(core_axis_name="c", subcore_axis_name="s"),
    scratch_types=[
        pltpu.VMEM((_C,), jnp.int32),
        pltpu.VMEM((_C,), jnp.int32),
        pltpu.VMEM((_C, _D), jnp.float32),
        pltpu.VMEM((_C, _CW), jnp.float32),
        pltpu.VMEM_SHARED((_N, _D), jnp.float32),
        pltpu.VMEM_SHARED((_N, _CW), jnp.float32),
        pltpu.SemaphoreType.DMA,
    ],
)(_sc_agg_body)


_R = 1000


def _mm2_body(x_ref, wl_ref, wr_ref, xl_ref, xr_ref):
    xb = x_ref[...]
    xl_ref[...] = lax.dot_general(xb, wl_ref[...], (((1,), (0,)), ((), ())),
                                  precision=_PREC,
                                  preferred_element_type=jnp.float32)
    xr_ref[...] = lax.dot_general(xb, wr_ref[...], (((1,), (0,)), ((), ())),
                                  precision=_PREC,
                                  preferred_element_type=jnp.float32)


def _tc_mm2(x, wl_t, wr_t):
    return pl.pallas_call(
        _mm2_body,
        grid=(_N // _R,),
        in_specs=[
            pl.BlockSpec((_R, _D), lambda i: (i, 0)),
            pl.BlockSpec((_D, _D), lambda i: (0, 0)),
            pl.BlockSpec((_D, _D), lambda i: (0, 0)),
        ],
        out_specs=[
            pl.BlockSpec((_R, _D), lambda i: (i, 0)),
            pl.BlockSpec((_R, _D), lambda i: (i, 0)),
        ],
        out_shape=[jax.ShapeDtypeStruct((_N, _D), jnp.float32),
                   jax.ShapeDtypeStruct((_N, _D), jnp.float32)],
    )(x, wl_t, wr_t)


def _mean_relu_mm2_body(acc_ref, cnt_ref, xr_ref, wl_ref, wr_ref,
                        xl2_ref, xr2_ref):
    a = acc_ref[0] + acc_ref[1]
    cn = cnt_ref[0, :, 0:1] + cnt_ref[1, :, 0:1]
    h = a / jnp.maximum(cn, 1.0) + xr_ref[...]
    h = jnp.maximum(h, 0.0)
    xl2_ref[...] = lax.dot_general(h, wl_ref[...], (((1,), (0,)), ((), ())),
                                   precision=_PREC,
                                   preferred_element_type=jnp.float32)
    xr2_ref[...] = lax.dot_general(h, wr_ref[...], (((1,), (0,)), ((), ())),
                                   precision=_PREC,
                                   preferred_element_type=jnp.float32)


def _tc_mean_relu_mm2(acc, cnt, xr, wl_t, wr_t):
    return pl.pallas_call(
        _mean_relu_mm2_body,
        grid=(_N // _R,),
        in_specs=[
            pl.BlockSpec((_NC, _R, _D), lambda i: (0, i, 0)),
            pl.BlockSpec((_NC, _R, _CW), lambda i: (0, i, 0)),
            pl.BlockSpec((_R, _D), lambda i: (i, 0)),
            pl.BlockSpec((_D, _D), lambda i: (0, 0)),
            pl.BlockSpec((_D, _D), lambda i: (0, 0)),
        ],
        out_specs=[
            pl.BlockSpec((_R, _D), lambda i: (i, 0)),
            pl.BlockSpec((_R, _D), lambda i: (i, 0)),
        ],
        out_shape=[jax.ShapeDtypeStruct((_N, _D), jnp.float32),
                   jax.ShapeDtypeStruct((_N, _D), jnp.float32)],
    )(acc, cnt, xr, wl_t, wr_t)


def _mean_lsm_body(acc_ref, cnt_ref, xr_ref, out_ref):
    a = acc_ref[0] + acc_ref[1]
    cn = cnt_ref[0, :, 0:1] + cnt_ref[1, :, 0:1]
    v = a / jnp.maximum(cn, 1.0) + xr_ref[...]
    m = jnp.max(v, axis=1, keepdims=True)
    z = v - m
    lse = jnp.log(jnp.sum(jnp.exp(z), axis=1, keepdims=True))
    out_ref[...] = z - lse


def _tc_mean_lsm(acc, cnt, xr):
    return pl.pallas_call(
        _mean_lsm_body,
        grid=(_N // _R,),
        in_specs=[
            pl.BlockSpec((_NC, _R, _D), lambda i: (0, i, 0)),
            pl.BlockSpec((_NC, _R, _CW), lambda i: (0, i, 0)),
            pl.BlockSpec((_R, _D), lambda i: (i, 0)),
        ],
        out_specs=pl.BlockSpec((_R, _D), lambda i: (i, 0)),
        out_shape=jax.ShapeDtypeStruct((_N, _D), jnp.float32),
    )(acc, cnt, xr)


def kernel(x, edge_index1, edge_index2, W1_l, W1_r, W2_l, W2_r):
    src1, dst1 = edge_index1[0], edge_index1[1]
    src2, dst2 = edge_index2[0], edge_index2[1]
    zrow = jnp.zeros((_RPT, _D), jnp.float32)
    zcnt = jnp.zeros((_RPT, _CW), jnp.float32)
    ones = jnp.ones((_C, _CW), jnp.float32)

    xl1, xr1 = _tc_mm2(x, W1_l.T, W1_r.T)
    acc1, cnt1 = _sc_agg(xl1, src1, dst1, zrow, zcnt, ones)
    xl2, xr2 = _tc_mean_relu_mm2(acc1, cnt1, xr1, W2_l.T, W2_r.T)
    acc2, cnt2 = _sc_agg(xl2, src2, dst2, zrow, zcnt, ones)
    return _tc_mean_lsm(acc2, cnt2, xr2)


# speedup vs baseline: 10.6837x; 10.6837x over previous
import functools

import jax
import jax.numpy as jnp
from jax import lax
from jax.experimental import pallas as pl
from jax.experimental.pallas import tpu as pltpu
from jax.experimental.pallas import tpu_sc as plsc

_N = 10000
_E = 320000
_D = 128
_NC = 2
_NS = 16
_NW = _NC * _NS
_EPW = _E // _NW
_C = 80
_K = _EPW // _C
_RPT = 624
_TAIL = _N - _NS * _RPT

_PREC = lax.Precision.HIGHEST


def _sc_agg_body(table, src, dst3, zrow, zcnt, ones,
                 out_acc, out_cnt0, out_cnt1, src_a, dst_a, rows0, rows1,
                 ones_v, acc_sh, cnt_sh, sem0, sem1, sem_c):
    c = lax.axis_index("c")
    s = lax.axis_index("s")
    w = c * _NS + s
    pltpu.sync_copy(src.at[pl.ds(w * _EPW, _EPW)], src_a)
    pltpu.sync_copy(dst3.at[w], dst_a)
    pltpu.sync_copy(zrow.at[pl.ds(0, _RPT)], acc_sh.at[pl.ds(s * _RPT, _RPT)])

    @pl.when(s == _NS - 1)
    def _():
        pltpu.sync_copy(zrow.at[pl.ds(0, _TAIL)],
                        acc_sh.at[pl.ds(_NS * _RPT, _TAIL)])

    @pl.when(s == 0)
    def _():
        pltpu.sync_copy(zcnt, cnt_sh)

    pltpu.sync_copy(ones, ones_v)
    plsc.subcore_barrier()

    def sidx(k):
        return src_a.at[pl.ds(k * _C, _C)]

    pltpu.async_copy(table.at[sidx(0)], rows0, sem0)

    def cnt_wait(k):
        pltpu.make_async_copy(ones_v, cnt_sh.at[dst_a.at[k]], sem_c).wait()

    def pair(j, carry):
        k0 = 2 * j
        k1 = k0 + 1
        pltpu.async_copy(table.at[sidx(k1)], rows1, sem1)
        pltpu.make_async_copy(table.at[sidx(k0)], rows0, sem0).wait()
        pltpu.sync_copy(rows0, acc_sh.at[dst_a.at[k0]], add=True)
        pltpu.async_copy(ones_v, cnt_sh.at[dst_a.at[k0]], sem_c, add=True)
        pltpu.async_copy(table.at[sidx(k0 + 2)], rows0, sem0)
        pltpu.make_async_copy(table.at[sidx(k1)], rows1, sem1).wait()
        pltpu.sync_copy(rows1, acc_sh.at[dst_a.at[k1]], add=True)
        pltpu.async_copy(ones_v, cnt_sh.at[dst_a.at[k1]], sem_c, add=True)

        @pl.when(j > 0)
        def _():
            cnt_wait(k0 - 2)
            cnt_wait(k0 - 1)

        return carry

    lax.fori_loop(0, _K // 2, pair, 0)
    pltpu.make_async_copy(table.at[sidx(_K - 1)], rows0, sem0).wait()
    pltpu.sync_copy(rows0, acc_sh.at[dst_a.at[_K - 1]], add=True)
    pltpu.async_copy(ones_v, cnt_sh.at[dst_a.at[_K - 1]], sem_c, add=True)
    cnt_wait(_K - 3)
    cnt_wait(_K - 2)
    cnt_wait(_K - 1)
    plsc.subcore_barrier()
    pltpu.sync_copy(acc_sh.at[pl.ds(s * _RPT, _RPT)],
                    out_acc.at[c, pl.ds(s * _RPT, _RPT)])

    @pl.when(s == _NS - 1)
    def _():
        pltpu.sync_copy(acc_sh.at[pl.ds(_NS * _RPT, _TAIL)],
                        out_acc.at[c, pl.ds(_NS * _RPT, _TAIL)])

    @pl.when(jnp.logical_and(s == 0, c == 0))
    def _():
        pltpu.sync_copy(cnt_sh, out_cnt0)

    @pl.when(jnp.logical_and(s == 0, c == 1))
    def _():
        pltpu.sync_copy(cnt_sh, out_cnt1)


@functools.cache
def _make_sc_agg():
    return functools.partial(
        pl.kernel,
        out_type=(jax.ShapeDtypeStruct((_NC, _N, _D), jnp.float32),
                  jax.ShapeDtypeStruct((_N,), jnp.float32),
                  jax.ShapeDtypeStruct((_N,), jnp.float32)),
        mesh=plsc.VectorSubcoreMesh(core_axis_name="c", subcore_axis_name="s",
                                    num_cores=_NC, num_subcores=_NS),
        scratch_types=[
            pltpu.VMEM((_EPW,), jnp.int32),
            pltpu.VMEM((_K, _C), jnp.int32),
            pltpu.VMEM((_C, _D), jnp.float32),
            pltpu.VMEM((_C, _D), jnp.float32),
            pltpu.VMEM((_C,), jnp.float32),
            pltpu.VMEM_SHARED((_N, _D), jnp.float32),
            pltpu.VMEM_SHARED((_N,), jnp.float32),
            pltpu.SemaphoreType.DMA,
            pltpu.SemaphoreType.DMA,
            pltpu.SemaphoreType.DMA,
        ],
    )(_sc_agg_body)


def _sc_agg(*args):
    return _make_sc_agg()(*args)


_R = 1000


def _mm1_body(x_ref, w_ref, out_ref):
    out_ref[...] = lax.dot_general(x_ref[...], w_ref[...],
                                   (((1,), (0,)), ((), ())),
                                   precision=_PREC,
                                   preferred_element_type=jnp.float32)


def _tc_mm(x, w_t):
    return pl.pallas_call(
        _mm1_body,
        grid=(_N // _R,),
        in_specs=[
            pl.BlockSpec((_R, _D), lambda i: (i, 0)),
            pl.BlockSpec((_D, _D), lambda i: (0, 0)),
        ],
        out_specs=pl.BlockSpec((_R, _D), lambda i: (i, 0)),
        out_shape=jax.ShapeDtypeStruct((_N, _D), jnp.float32),
    )(x, w_t)


def _mean_relu_mm2_body(acc_ref, cnt0_ref, cnt1_ref, xr_ref, wl_ref, wr_ref,
                        xl2_ref, xr2_ref):
    a = acc_ref[0] + acc_ref[1]
    cn = cnt0_ref[...] + cnt1_ref[...]
    h = a / jnp.maximum(cn, 1.0) + xr_ref[...]
    h = jnp.maximum(h, 0.0)
    xl2_ref[...] = lax.dot_general(h, wl_ref[...], (((1,), (0,)), ((), ())),
                                   precision=_PREC,
                                   preferred_element_type=jnp.float32)
    xr2_ref[...] = lax.dot_general(h, wr_ref[...], (((1,), (0,)), ((), ())),
                                   precision=_PREC,
                                   preferred_element_type=jnp.float32)


def _tc_mean_relu_mm2(acc, cnt0, cnt1, xr, wl_t, wr_t):
    return pl.pallas_call(
        _mean_relu_mm2_body,
        grid=(_N // _R,),
        in_specs=[
            pl.BlockSpec((_NC, _R, _D), lambda i: (0, i, 0)),
            pl.BlockSpec((_R, 1), lambda i: (i, 0)),
            pl.BlockSpec((_R, 1), lambda i: (i, 0)),
            pl.BlockSpec((_R, _D), lambda i: (i, 0)),
            pl.BlockSpec((_D, _D), lambda i: (0, 0)),
            pl.BlockSpec((_D, _D), lambda i: (0, 0)),
        ],
        out_specs=[
            pl.BlockSpec((_R, _D), lambda i: (i, 0)),
            pl.BlockSpec((_R, _D), lambda i: (i, 0)),
        ],
        out_shape=[jax.ShapeDtypeStruct((_N, _D), jnp.float32),
                   jax.ShapeDtypeStruct((_N, _D), jnp.float32)],
    )(acc, cnt0, cnt1, xr, wl_t, wr_t)


def _mean_lsm_body(acc_ref, cnt0_ref, cnt1_ref, xr_ref, out_ref):
    a = acc_ref[0] + acc_ref[1]
    cn = cnt0_ref[...] + cnt1_ref[...]
    v = a / jnp.maximum(cn, 1.0) + xr_ref[...]
    m = jnp.max(v, axis=1, keepdims=True)
    z = v - m
    lse = jnp.log(jnp.sum(jnp.exp(z), axis=1, keepdims=True))
    out_ref[...] = z - lse


def _tc_mean_lsm(acc, cnt0, cnt1, xr):
    return pl.pallas_call(
        _mean_lsm_body,
        grid=(_N // _R,),
        in_specs=[
            pl.BlockSpec((_NC, _R, _D), lambda i: (0, i, 0)),
            pl.BlockSpec((_R, 1), lambda i: (i, 0)),
            pl.BlockSpec((_R, 1), lambda i: (i, 0)),
            pl.BlockSpec((_R, _D), lambda i: (i, 0)),
        ],
        out_specs=pl.BlockSpec((_R, _D), lambda i: (i, 0)),
        out_shape=jax.ShapeDtypeStruct((_N, _D), jnp.float32),
    )(acc, cnt0, cnt1, xr)


def kernel(x, edge_index1, edge_index2, W1_l, W1_r, W2_l, W2_r):
    src1 = edge_index1[0]
    dst1 = edge_index1[1].reshape(_NW, _K, _C)
    src2 = edge_index2[0]
    dst2 = edge_index2[1].reshape(_NW, _K, _C)
    zrow = jnp.zeros((_RPT, _D), jnp.float32)
    zcnt = jnp.zeros((_N,), jnp.float32)
    ones = jnp.ones((_C,), jnp.float32)

    xl1 = _tc_mm(x, W1_l.T)
    acc1, cnt1a, cnt1b = _sc_agg(xl1, src1, dst1, zrow, zcnt, ones)
    xr1 = _tc_mm(x, W1_r.T)
    xl2, xr2 = _tc_mean_relu_mm2(acc1, cnt1a.reshape(_N, 1),
                                 cnt1b.reshape(_N, 1), xr1, W2_l.T, W2_r.T)
    acc2, cnt2a, cnt2b = _sc_agg(xl2, src2, dst2, zrow, zcnt, ones)
    return _tc_mean_lsm(acc2, cnt2a.reshape(_N, 1), cnt2b.reshape(_N, 1), xr2)


# speedup vs baseline: 10.7609x; 1.0072x over previous
import functools

import jax
import jax.numpy as jnp
from jax import lax
from jax.experimental import pallas as pl
from jax.experimental.pallas import tpu as pltpu
from jax.experimental.pallas import tpu_sc as plsc

_N = 10000
_E = 320000
_D = 128
_NC = 2
_NS = 16
_NW = _NC * _NS
_EPW = _E // _NW
_C = 80
_K = _EPW // _C
_RPT = 624
_TAIL = _N - _NS * _RPT

_PREC = lax.Precision.HIGHEST


def _sc_agg_body(table, src, dst3, zrow, zcnt, ones,
                 out_acc, out_cnt0, out_cnt1, src_a, dst_a, rows0, rows1,
                 ones_v, acc_sh, cnt_sh, sem0, sem1, sem_c):
    c = lax.axis_index("c")
    s = lax.axis_index("s")
    w = c * _NS + s
    pltpu.sync_copy(src.at[pl.ds(w * _EPW, _EPW)], src_a)
    pltpu.sync_copy(dst3.at[w], dst_a)
    pltpu.sync_copy(zrow.at[pl.ds(0, _RPT)], acc_sh.at[pl.ds(s * _RPT, _RPT)])

    @pl.when(s == _NS - 1)
    def _():
        pltpu.sync_copy(zrow.at[pl.ds(0, _TAIL)],
                        acc_sh.at[pl.ds(_NS * _RPT, _TAIL)])

    @pl.when(s == 0)
    def _():
        pltpu.sync_copy(zcnt, cnt_sh)

    pltpu.sync_copy(ones, ones_v)
    plsc.subcore_barrier()

    def sidx(k):
        return src_a.at[pl.ds(k * _C, _C)]

    pltpu.async_copy(table.at[sidx(0)], rows0, sem0)

    def cnt_wait(k):
        pltpu.make_async_copy(ones_v, cnt_sh.at[dst_a.at[k]], sem_c).wait()

    def pair(j, carry):
        k0 = 2 * j
        k1 = k0 + 1
        pltpu.async_copy(table.at[sidx(k1)], rows1, sem1)
        pltpu.make_async_copy(table.at[sidx(k0)], rows0, sem0).wait()
        pltpu.sync_copy(rows0, acc_sh.at[dst_a.at[k0]], add=True)
        pltpu.async_copy(ones_v, cnt_sh.at[dst_a.at[k0]], sem_c, add=True)
        pltpu.async_copy(table.at[sidx(k0 + 2)], rows0, sem0)
        pltpu.make_async_copy(table.at[sidx(k1)], rows1, sem1).wait()
        pltpu.sync_copy(rows1, acc_sh.at[dst_a.at[k1]], add=True)
        pltpu.async_copy(ones_v, cnt_sh.at[dst_a.at[k1]], sem_c, add=True)

        @pl.when(j > 0)
        def _():
            cnt_wait(k0 - 2)
            cnt_wait(k0 - 1)

        return carry

    lax.fori_loop(0, _K // 2, pair, 0)
    pltpu.make_async_copy(table.at[sidx(_K - 1)], rows0, sem0).wait()
    pltpu.sync_copy(rows0, acc_sh.at[dst_a.at[_K - 1]], add=True)
    pltpu.async_copy(ones_v, cnt_sh.at[dst_a.at[_K - 1]], sem_c, add=True)
    cnt_wait(_K - 3)
    cnt_wait(_K - 2)
    cnt_wait(_K - 1)
    plsc.subcore_barrier()
    pltpu.sync_copy(acc_sh.at[pl.ds(s * _RPT, _RPT)],
                    out_acc.at[c, pl.ds(s * _RPT, _RPT)])

    @pl.when(s == _NS - 1)
    def _():
        pltpu.sync_copy(acc_sh.at[pl.ds(_NS * _RPT, _TAIL)],
                        out_acc.at[c, pl.ds(_NS * _RPT, _TAIL)])

    @pl.when(jnp.logical_and(s == 0, c == 0))
    def _():
        pltpu.sync_copy(cnt_sh, out_cnt0)

    @pl.when(jnp.logical_and(s == 0, c == 1))
    def _():
        pltpu.sync_copy(cnt_sh, out_cnt1)


@functools.cache
def _make_sc_agg():
    return functools.partial(
        pl.kernel,
        out_type=(jax.ShapeDtypeStruct((_NC, _N, _D), jnp.float32),
                  jax.ShapeDtypeStruct((_N,), jnp.float32),
                  jax.ShapeDtypeStruct((_N,), jnp.float32)),
        mesh=plsc.VectorSubcoreMesh(core_axis_name="c", subcore_axis_name="s",
                                    num_cores=_NC, num_subcores=_NS),
        scratch_types=[
            pltpu.VMEM((_EPW,), jnp.int32),
            pltpu.VMEM((_K, _C), jnp.int32),
            pltpu.VMEM((_C, _D), jnp.float32),
            pltpu.VMEM((_C, _D), jnp.float32),
            pltpu.VMEM((_C,), jnp.float32),
            pltpu.VMEM_SHARED((_N, _D), jnp.float32),
            pltpu.VMEM_SHARED((_N,), jnp.float32),
            pltpu.SemaphoreType.DMA,
            pltpu.SemaphoreType.DMA,
            pltpu.SemaphoreType.DMA,
        ],
    )(_sc_agg_body)


def _sc_agg(*args):
    return _make_sc_agg()(*args)


_R = 1000


def _mm1_body(x_ref, w_ref, out_ref):
    out_ref[...] = lax.dot_general(x_ref[...], w_ref[...],
                                   (((1,), (0,)), ((), ())),
                                   precision=_PREC,
                                   preferred_element_type=jnp.float32)


def _tc_mm(x, w_t):
    return pl.pallas_call(
        _mm1_body,
        grid=(_N // _R,),
        in_specs=[
            pl.BlockSpec((_R, _D), lambda i: (i, 0)),
            pl.BlockSpec((_D, _D), lambda i: (0, 0)),
        ],
        out_specs=pl.BlockSpec((_R, _D), lambda i: (i, 0)),
        out_shape=jax.ShapeDtypeStruct((_N, _D), jnp.float32),
    )(x, w_t)


def _mean_relu_mm_body(acc_ref, cnt0_ref, cnt1_ref, xr_ref, wl_ref,
                       h_ref, xl2_ref):
    a = acc_ref[0] + acc_ref[1]
    cn = cnt0_ref[...] + cnt1_ref[...]
    h = a / jnp.maximum(cn, 1.0) + xr_ref[...]
    h = jnp.maximum(h, 0.0)
    h_ref[...] = h
    xl2_ref[...] = lax.dot_general(h, wl_ref[...], (((1,), (0,)), ((), ())),
                                   precision=_PREC,
                                   preferred_element_type=jnp.float32)


def _tc_mean_relu_mm(acc, cnt0, cnt1, xr, wl_t):
    return pl.pallas_call(
        _mean_relu_mm_body,
        grid=(_N // _R,),
        in_specs=[
            pl.BlockSpec((_NC, _R, _D), lambda i: (0, i, 0)),
            pl.BlockSpec((_R, 1), lambda i: (i, 0)),
            pl.BlockSpec((_R, 1), lambda i: (i, 0)),
            pl.BlockSpec((_R, _D), lambda i: (i, 0)),
            pl.BlockSpec((_D, _D), lambda i: (0, 0)),
        ],
        out_specs=[
            pl.BlockSpec((_R, _D), lambda i: (i, 0)),
            pl.BlockSpec((_R, _D), lambda i: (i, 0)),
        ],
        out_shape=[jax.ShapeDtypeStruct((_N, _D), jnp.float32),
                   jax.ShapeDtypeStruct((_N, _D), jnp.float32)],
    )(acc, cnt0, cnt1, xr, wl_t)


def _mean_lsm_body(acc_ref, cnt0_ref, cnt1_ref, xr_ref, out_ref):
    a = acc_ref[0] + acc_ref[1]
    cn = cnt0_ref[...] + cnt1_ref[...]
    v = a / jnp.maximum(cn, 1.0) + xr_ref[...]
    m = jnp.max(v, axis=1, keepdims=True)
    z = v - m
    lse = jnp.log(jnp.sum(jnp.exp(z), axis=1, keepdims=True))
    out_ref[...] = z - lse


def _tc_mean_lsm(acc, cnt0, cnt1, xr):
    return pl.pallas_call(
        _mean_lsm_body,
        grid=(_N // _R,),
        in_specs=[
            pl.BlockSpec((_NC, _R, _D), lambda i: (0, i, 0)),
            pl.BlockSpec((_R, 1), lambda i: (i, 0)),
            pl.BlockSpec((_R, 1), lambda i: (i, 0)),
            pl.BlockSpec((_R, _D), lambda i: (i, 0)),
        ],
        out_specs=pl.BlockSpec((_R, _D), lambda i: (i, 0)),
        out_shape=jax.ShapeDtypeStruct((_N, _D), jnp.float32),
    )(acc, cnt0, cnt1, xr)


def kernel(x, edge_index1, edge_index2, W1_l, W1_r, W2_l, W2_r):
    src1 = edge_index1[0]
    dst1 = edge_index1[1].reshape(_NW, _K, _C)
    src2 = edge_index2[0]
    dst2 = edge_index2[1].reshape(_NW, _K, _C)
    zrow = jnp.zeros((_RPT, _D), jnp.float32)
    zcnt = jnp.zeros((_N,), jnp.float32)
    ones = jnp.ones((_C,), jnp.float32)

    xl1 = _tc_mm(x, W1_l.T)
    acc1, cnt1a, cnt1b = _sc_agg(xl1, src1, dst1, zrow, zcnt, ones)
    xr1 = _tc_mm(x, W1_r.T)
    h, xl2 = _tc_mean_relu_mm(acc1, cnt1a.reshape(_N, 1),
                              cnt1b.reshape(_N, 1), xr1, W2_l.T)
    acc2, cnt2a, cnt2b = _sc_agg(xl2, src2, dst2, zrow, zcnt, ones)
    xr2 = _tc_mm(h, W2_r.T)
    return _tc_mean_lsm(acc2, cnt2a.reshape(_N, 1), cnt2b.reshape(_N, 1), xr2)


# speedup vs baseline: 10.9213x; 1.0149x over previous
import functools

import jax
import jax.numpy as jnp
from jax import lax
from jax.experimental import pallas as pl
from jax.experimental.pallas import tpu as pltpu
from jax.experimental.pallas import tpu_sc as plsc

_N = 10000
_E = 320000
_D = 128
_NC = 2
_NS = 16
_NW = _NC * _NS
_EPW = _E // _NW
_C = 80
_K = _EPW // _C
_RPT = 624
_TAIL = _N - _NS * _RPT

_PREC = lax.Precision.HIGHEST


def _sc_agg_body(table, src, dst3, zrow, zcnt, ones,
                 out_acc, out_cnt0, out_cnt1, src_a, dst_a, rows0, rows1,
                 ones_v, acc_sh, cnt_sh, sem0, sem1, sem_c, sem_i):
    c = lax.axis_index("c")
    s = lax.axis_index("s")
    w = c * _NS + s
    pltpu.async_copy(src.at[pl.ds(w * _EPW, _EPW)], src_a, sem_i)
    pltpu.async_copy(dst3.at[w], dst_a, sem_i)
    pltpu.async_copy(zrow.at[pl.ds(0, _RPT)],
                     acc_sh.at[pl.ds(s * _RPT, _RPT)], sem_i)
    pltpu.async_copy(ones, ones_v, sem_i)

    @pl.when(s == _NS - 1)
    def _():
        pltpu.async_copy(zrow.at[pl.ds(0, _TAIL)],
                        acc_sh.at[pl.ds(_NS * _RPT, _TAIL)], sem_i)

    @pl.when(s == 0)
    def _():
        pltpu.async_copy(zcnt, cnt_sh, sem_i)

    def sidx(k):
        return src_a.at[pl.ds(k * _C, _C)]

    pltpu.make_async_copy(src.at[pl.ds(w * _EPW, _EPW)], src_a, sem_i).wait()
    pltpu.async_copy(table.at[sidx(0)], rows0, sem0)
    pltpu.make_async_copy(dst3.at[w], dst_a, sem_i).wait()
    pltpu.make_async_copy(zrow.at[pl.ds(0, _RPT)],
                          acc_sh.at[pl.ds(s * _RPT, _RPT)], sem_i).wait()
    pltpu.make_async_copy(ones, ones_v, sem_i).wait()

    @pl.when(s == _NS - 1)
    def _():
        pltpu.make_async_copy(zrow.at[pl.ds(0, _TAIL)],
                              acc_sh.at[pl.ds(_NS * _RPT, _TAIL)], sem_i).wait()

    @pl.when(s == 0)
    def _():
        pltpu.make_async_copy(zcnt, cnt_sh, sem_i).wait()

    plsc.subcore_barrier()


    def cnt_wait(k):
        pltpu.make_async_copy(ones_v, cnt_sh.at[dst_a.at[k]], sem_c).wait()

    def pair(j, carry):
        k0 = 2 * j
        k1 = k0 + 1
        pltpu.async_copy(table.at[sidx(k1)], rows1, sem1)
        pltpu.make_async_copy(table.at[sidx(k0)], rows0, sem0).wait()
        pltpu.sync_copy(rows0, acc_sh.at[dst_a.at[k0]], add=True)
        pltpu.async_copy(ones_v, cnt_sh.at[dst_a.at[k0]], sem_c, add=True)
        pltpu.async_copy(table.at[sidx(k0 + 2)], rows0, sem0)
        pltpu.make_async_copy(table.at[sidx(k1)], rows1, sem1).wait()
        pltpu.sync_copy(rows1, acc_sh.at[dst_a.at[k1]], add=True)
        pltpu.async_copy(ones_v, cnt_sh.at[dst_a.at[k1]], sem_c, add=True)

        @pl.when(j > 0)
        def _():
            cnt_wait(k0 - 2)
            cnt_wait(k0 - 1)

        return carry

    lax.fori_loop(0, _K // 2, pair, 0)
    pltpu.make_async_copy(table.at[sidx(_K - 1)], rows0, sem0).wait()
    pltpu.sync_copy(rows0, acc_sh.at[dst_a.at[_K - 1]], add=True)
    pltpu.async_copy(ones_v, cnt_sh.at[dst_a.at[_K - 1]], sem_c, add=True)
    cnt_wait(_K - 3)
    cnt_wait(_K - 2)
    cnt_wait(_K - 1)
    plsc.subcore_barrier()
    pltpu.sync_copy(acc_sh.at[pl.ds(s * _RPT, _RPT)],
                    out_acc.at[c, pl.ds(s * _RPT, _RPT)])

    @pl.when(s == _NS - 1)
    def _():
        pltpu.sync_copy(acc_sh.at[pl.ds(_NS * _RPT, _TAIL)],
                        out_acc.at[c, pl.ds(_NS * _RPT, _TAIL)])

    @pl.when(jnp.logical_and(s == 0, c == 0))
    def _():
        pltpu.sync_copy(cnt_sh, out_cnt0)

    @pl.when(jnp.logical_and(s == 0, c == 1))
    def _():
        pltpu.sync_copy(cnt_sh, out_cnt1)


@functools.cache
def _make_sc_agg():
    return functools.partial(
        pl.kernel,
        out_type=(jax.ShapeDtypeStruct((_NC, _N, _D), jnp.float32),
                  jax.ShapeDtypeStruct((_N,), jnp.float32),
                  jax.ShapeDtypeStruct((_N,), jnp.float32)),
        mesh=plsc.VectorSubcoreMesh(core_axis_name="c", subcore_axis_name="s",
                                    num_cores=_NC, num_subcores=_NS),
        scratch_types=[
            pltpu.VMEM((_EPW,), jnp.int32),
            pltpu.VMEM((_K, _C), jnp.int32),
            pltpu.VMEM((_C, _D), jnp.float32),
            pltpu.VMEM((_C, _D), jnp.float32),
            pltpu.VMEM((_C,), jnp.float32),
            pltpu.VMEM_SHARED((_N, _D), jnp.float32),
            pltpu.VMEM_SHARED((_N,), jnp.float32),
            pltpu.SemaphoreType.DMA,
            pltpu.SemaphoreType.DMA,
            pltpu.SemaphoreType.DMA,
            pltpu.SemaphoreType.DMA,
        ],
    )(_sc_agg_body)


def _sc_agg(*args):
    return _make_sc_agg()(*args)


_R = 1000


def _mm1_body(x_ref, w_ref, out_ref):
    out_ref[...] = lax.dot_general(x_ref[...], w_ref[...],
                                   (((1,), (0,)), ((), ())),
                                   precision=_PREC,
                                   preferred_element_type=jnp.float32)


def _tc_mm(x, w_t):
    return pl.pallas_call(
        _mm1_body,
        grid=(_N // _R,),
        in_specs=[
            pl.BlockSpec((_R, _D), lambda i: (i, 0)),
            pl.BlockSpec((_D, _D), lambda i: (0, 0)),
        ],
        out_specs=pl.BlockSpec((_R, _D), lambda i: (i, 0)),
        out_shape=jax.ShapeDtypeStruct((_N, _D), jnp.float32),
    )(x, w_t)


def _mean_relu_mm_body(acc_ref, cnt0_ref, cnt1_ref, xr_ref, wl_ref,
                       h_ref, xl2_ref):
    a = acc_ref[0] + acc_ref[1]
    cn = cnt0_ref[...] + cnt1_ref[...]
    h = a / jnp.maximum(cn, 1.0) + xr_ref[...]
    h = jnp.maximum(h, 0.0)
    h_ref[...] = h
    xl2_ref[...] = lax.dot_general(h, wl_ref[...], (((1,), (0,)), ((), ())),
                                   precision=_PREC,
                                   preferred_element_type=jnp.float32)


def _tc_mean_relu_mm(acc, cnt0, cnt1, xr, wl_t):
    return pl.pallas_call(
        _mean_relu_mm_body,
        grid=(_N // _R,),
        in_specs=[
            pl.BlockSpec((_NC, _R, _D), lambda i: (0, i, 0)),
            pl.BlockSpec((_R, 1), lambda i: (i, 0)),
            pl.BlockSpec((_R, 1), lambda i: (i, 0)),
            pl.BlockSpec((_R, _D), lambda i: (i, 0)),
            pl.BlockSpec((_D, _D), lambda i: (0, 0)),
        ],
        out_specs=[
            pl.BlockSpec((_R, _D), lambda i: (i, 0)),
            pl.BlockSpec((_R, _D), lambda i: (i, 0)),
        ],
        out_shape=[jax.ShapeDtypeStruct((_N, _D), jnp.float32),
                   jax.ShapeDtypeStruct((_N, _D), jnp.float32)],
    )(acc, cnt0, cnt1, xr, wl_t)


def _mean_lsm_body(acc_ref, cnt0_ref, cnt1_ref, xr_ref, out_ref):
    a = acc_ref[0] + acc_ref[1]
    cn = cnt0_ref[...] + cnt1_ref[...]
    v = a / jnp.maximum(cn, 1.0) + xr_ref[...]
    m = jnp.max(v, axis=1, keepdims=True)
    z = v - m
    lse = jnp.log(jnp.sum(jnp.exp(z), axis=1, keepdims=True))
    out_ref[...] = z - lse


def _tc_mean_lsm(acc, cnt0, cnt1, xr):
    return pl.pallas_call(
        _mean_lsm_body,
        grid=(_N // _R,),
        in_specs=[
            pl.BlockSpec((_NC, _R, _D), lambda i: (0, i, 0)),
            pl.BlockSpec((_R, 1), lambda i: (i, 0)),
            pl.BlockSpec((_R, 1), lambda i: (i, 0)),
            pl.BlockSpec((_R, _D), lambda i: (i, 0)),
        ],
        out_specs=pl.BlockSpec((_R, _D), lambda i: (i, 0)),
        out_shape=jax.ShapeDtypeStruct((_N, _D), jnp.float32),
    )(acc, cnt0, cnt1, xr)


def kernel(x, edge_index1, edge_index2, W1_l, W1_r, W2_l, W2_r):
    src1 = edge_index1[0]
    dst1 = edge_index1[1].reshape(_NW, _K, _C)
    src2 = edge_index2[0]
    dst2 = edge_index2[1].reshape(_NW, _K, _C)
    zrow = jnp.zeros((_RPT, _D), jnp.float32)
    zcnt = jnp.zeros((_N,), jnp.float32)
    ones = jnp.ones((_C,), jnp.float32)

    xl1 = _tc_mm(x, W1_l.T)
    acc1, cnt1a, cnt1b = _sc_agg(xl1, src1, dst1, zrow, zcnt, ones)
    xr1 = _tc_mm(x, W1_r.T)
    h, xl2 = _tc_mean_relu_mm(acc1, cnt1a.reshape(_N, 1),
                              cnt1b.reshape(_N, 1), xr1, W2_l.T)
    acc2, cnt2a, cnt2b = _sc_agg(xl2, src2, dst2, zrow, zcnt, ones)
    xr2 = _tc_mm(h, W2_r.T)
    return _tc_mean_lsm(acc2, cnt2a.reshape(_N, 1), cnt2b.reshape(_N, 1), xr2)


# speedup vs baseline: 11.2413x; 1.0293x over previous
import functools

import jax
import jax.numpy as jnp
from jax import lax
from jax.experimental import pallas as pl
from jax.experimental.pallas import tpu as pltpu
from jax.experimental.pallas import tpu_sc as plsc

_N = 10000
_E = 320000
_D = 128
_NC = 2
_NS = 16
_NW = _NC * _NS
_EPW = _E // _NW
_C = 80
_K = _EPW // _C
_RPT = 624
_TAIL = _N - _NS * _RPT

_PREC = lax.Precision.HIGHEST


def _sc_agg_body(table, src, dst3, zrow, zcnt, ones,
                 out_acc, out_cnt0, out_cnt1, src_a, dst_a, rows0, rows1,
                 ones_v, acc_sh, cnt_sh, sem0, sem1, sem_c, sem_i):
    c = lax.axis_index("c")
    s = lax.axis_index("s")
    w = c * _NS + s
    pltpu.async_copy(src.at[pl.ds(w * _EPW, _EPW)], src_a, sem_i)
    pltpu.async_copy(dst3.at[w], dst_a, sem_i)
    pltpu.async_copy(zrow.at[pl.ds(0, _RPT)],
                     acc_sh.at[pl.ds(s * _RPT, _RPT)], sem_i)
    pltpu.async_copy(ones, ones_v, sem_i)

    @pl.when(s == _NS - 1)
    def _():
        pltpu.async_copy(zrow.at[pl.ds(0, _TAIL)],
                        acc_sh.at[pl.ds(_NS * _RPT, _TAIL)], sem_i)

    @pl.when(s == 0)
    def _():
        pltpu.async_copy(zcnt, cnt_sh, sem_i)

    def sidx(k):
        return src_a.at[pl.ds(k * _C, _C)]

    pltpu.make_async_copy(src.at[pl.ds(w * _EPW, _EPW)], src_a, sem_i).wait()
    pltpu.async_copy(table.at[sidx(0)], rows0, sem0)
    pltpu.make_async_copy(dst3.at[w], dst_a, sem_i).wait()
    pltpu.make_async_copy(zrow.at[pl.ds(0, _RPT)],
                          acc_sh.at[pl.ds(s * _RPT, _RPT)], sem_i).wait()
    pltpu.make_async_copy(ones, ones_v, sem_i).wait()

    @pl.when(s == _NS - 1)
    def _():
        pltpu.make_async_copy(zrow.at[pl.ds(0, _TAIL)],
                              acc_sh.at[pl.ds(_NS * _RPT, _TAIL)], sem_i).wait()

    @pl.when(s == 0)
    def _():
        pltpu.make_async_copy(zcnt, cnt_sh, sem_i).wait()

    plsc.subcore_barrier()


    def cnt_wait(k):
        pltpu.make_async_copy(ones_v, cnt_sh.at[dst_a.at[k]], sem_c).wait()

    def pair(j, carry):
        k0 = 2 * j
        k1 = k0 + 1
        pltpu.async_copy(table.at[sidx(k1)], rows1, sem1)
        pltpu.make_async_copy(table.at[sidx(k0)], rows0, sem0).wait()
        pltpu.sync_copy(rows0, acc_sh.at[dst_a.at[k0]], add=True)
        pltpu.async_copy(ones_v, cnt_sh.at[dst_a.at[k0]], sem_c, add=True)
        pltpu.async_copy(table.at[sidx(k0 + 2)], rows0, sem0)
        pltpu.make_async_copy(table.at[sidx(k1)], rows1, sem1).wait()
        pltpu.sync_copy(rows1, acc_sh.at[dst_a.at[k1]], add=True)
        pltpu.async_copy(ones_v, cnt_sh.at[dst_a.at[k1]], sem_c, add=True)

        @pl.when(j > 0)
        def _():
            cnt_wait(k0 - 2)
            cnt_wait(k0 - 1)

        return carry

    lax.fori_loop(0, _K // 2, pair, 0)
    pltpu.make_async_copy(table.at[sidx(_K - 1)], rows0, sem0).wait()
    pltpu.sync_copy(rows0, acc_sh.at[dst_a.at[_K - 1]], add=True)
    pltpu.async_copy(ones_v, cnt_sh.at[dst_a.at[_K - 1]], sem_c, add=True)
    cnt_wait(_K - 3)
    cnt_wait(_K - 2)
    cnt_wait(_K - 1)
    plsc.subcore_barrier()
    pltpu.sync_copy(acc_sh.at[pl.ds(s * _RPT, _RPT)],
                    out_acc.at[c, pl.ds(s * _RPT, _RPT)])

    @pl.when(s == _NS - 1)
    def _():
        pltpu.sync_copy(acc_sh.at[pl.ds(_NS * _RPT, _TAIL)],
                        out_acc.at[c, pl.ds(_NS * _RPT, _TAIL)])

    @pl.when(jnp.logical_and(s == 0, c == 0))
    def _():
        pltpu.sync_copy(cnt_sh, out_cnt0)

    @pl.when(jnp.logical_and(s == 0, c == 1))
    def _():
        pltpu.sync_copy(cnt_sh, out_cnt1)


@functools.cache
def _make_sc_agg():
    return functools.partial(
        pl.kernel,
        out_type=(jax.ShapeDtypeStruct((_NC, _N, _D), jnp.float32),
                  jax.ShapeDtypeStruct((_N,), jnp.float32),
                  jax.ShapeDtypeStruct((_N,), jnp.float32)),
        mesh=plsc.VectorSubcoreMesh(core_axis_name="c", subcore_axis_name="s",
                                    num_cores=_NC, num_subcores=_NS),
        scratch_types=[
            pltpu.VMEM((_EPW,), jnp.int32),
            pltpu.VMEM((_K, _C), jnp.int32),
            pltpu.VMEM((_C, _D), jnp.float32),
            pltpu.VMEM((_C, _D), jnp.float32),
            pltpu.VMEM((_C,), jnp.float32),
            pltpu.VMEM_SHARED((_N, _D), jnp.float32),
            pltpu.VMEM_SHARED((_N,), jnp.float32),
            pltpu.SemaphoreType.DMA,
            pltpu.SemaphoreType.DMA,
            pltpu.SemaphoreType.DMA,
            pltpu.SemaphoreType.DMA,
        ],
    )(_sc_agg_body)


def _sc_agg(*args):
    return _make_sc_agg()(*args)


_R = 2000


def _mm1_body(x_ref, w_ref, out_ref):
    out_ref[...] = lax.dot_general(x_ref[...], w_ref[...],
                                   (((1,), (0,)), ((), ())),
                                   precision=_PREC,
                                   preferred_element_type=jnp.float32)


def _tc_mm(x, w_t):
    return pl.pallas_call(
        _mm1_body,
        grid=(_N // _R,),
        in_specs=[
            pl.BlockSpec((_R, _D), lambda i: (i, 0)),
            pl.BlockSpec((_D, _D), lambda i: (0, 0)),
        ],
        out_specs=pl.BlockSpec((_R, _D), lambda i: (i, 0)),
        out_shape=jax.ShapeDtypeStruct((_N, _D), jnp.float32),
    )(x, w_t)


def _mean_relu_mm_body(acc_ref, cnt0_ref, cnt1_ref, xr_ref, wl_ref,
                       h_ref, xl2_ref):
    a = acc_ref[0] + acc_ref[1]
    cn = cnt0_ref[...] + cnt1_ref[...]
    h = a / jnp.maximum(cn, 1.0) + xr_ref[...]
    h = jnp.maximum(h, 0.0)
    h_ref[...] = h
    xl2_ref[...] = lax.dot_general(h, wl_ref[...], (((1,), (0,)), ((), ())),
                                   precision=_PREC,
                                   preferred_element_type=jnp.float32)


def _tc_mean_relu_mm(acc, cnt0, cnt1, xr, wl_t):
    return pl.pallas_call(
        _mean_relu_mm_body,
        grid=(_N // _R,),
        in_specs=[
            pl.BlockSpec((_NC, _R, _D), lambda i: (0, i, 0)),
            pl.BlockSpec((_R, 1), lambda i: (i, 0)),
            pl.BlockSpec((_R, 1), lambda i: (i, 0)),
            pl.BlockSpec((_R, _D), lambda i: (i, 0)),
            pl.BlockSpec((_D, _D), lambda i: (0, 0)),
        ],
        out_specs=[
            pl.BlockSpec((_R, _D), lambda i: (i, 0)),
            pl.BlockSpec((_R, _D), lambda i: (i, 0)),
        ],
        out_shape=[jax.ShapeDtypeStruct((_N, _D), jnp.float32),
                   jax.ShapeDtypeStruct((_N, _D), jnp.float32)],
    )(acc, cnt0, cnt1, xr, wl_t)


def _mean_lsm_body(acc_ref, cnt0_ref, cnt1_ref, xr_ref, out_ref):
    a = acc_ref[0] + acc_ref[1]
    cn = cnt0_ref[...] + cnt1_ref[...]
    v = a / jnp.maximum(cn, 1.0) + xr_ref[...]
    m = jnp.max(v, axis=1, keepdims=True)
    z = v - m
    lse = jnp.log(jnp.sum(jnp.exp(z), axis=1, keepdims=True))
    out_ref[...] = z - lse


def _tc_mean_lsm(acc, cnt0, cnt1, xr):
    return pl.pallas_call(
        _mean_lsm_body,
        grid=(_N // _R,),
        in_specs=[
            pl.BlockSpec((_NC, _R, _D), lambda i: (0, i, 0)),
            pl.BlockSpec((_R, 1), lambda i: (i, 0)),
            pl.BlockSpec((_R, 1), lambda i: (i, 0)),
            pl.BlockSpec((_R, _D), lambda i: (i, 0)),
        ],
        out_specs=pl.BlockSpec((_R, _D), lambda i: (i, 0)),
        out_shape=jax.ShapeDtypeStruct((_N, _D), jnp.float32),
    )(acc, cnt0, cnt1, xr)


def kernel(x, edge_index1, edge_index2, W1_l, W1_r, W2_l, W2_r):
    src1 = edge_index1[0]
    dst1 = edge_index1[1].reshape(_NW, _K, _C)
    src2 = edge_index2[0]
    dst2 = edge_index2[1].reshape(_NW, _K, _C)
    zrow = jnp.zeros((_RPT, _D), jnp.float32)
    zcnt = jnp.zeros((_N,), jnp.float32)
    ones = jnp.ones((_C,), jnp.float32)

    xl1 = _tc_mm(x, W1_l.T)
    acc1, cnt1a, cnt1b = _sc_agg(xl1, src1, dst1, zrow, zcnt, ones)
    xr1 = _tc_mm(x, W1_r.T)
    h, xl2 = _tc_mean_relu_mm(acc1, cnt1a.reshape(_N, 1),
                              cnt1b.reshape(_N, 1), xr1, W2_l.T)
    acc2, cnt2a, cnt2b = _sc_agg(xl2, src2, dst2, zrow, zcnt, ones)
    xr2 = _tc_mm(h, W2_r.T)
    return _tc_mean_lsm(acc2, cnt2a.reshape(_N, 1), cnt2b.reshape(_N, 1), xr2)
